# baseline (speedup 1.0000x reference)
import jax
import jax.numpy as jnp
from jax.experimental import pallas as pl

B = 16384
INPUT_DIM = 512
EMB_DIM = 128
N_CLUSTERS = 64

BM = 1024


def _fused_body(x_ref, w_ref, c_ref, out_ref):
    xb = x_ref[...]
    w = w_ref[...]
    c = c_ref[...]
    xe = jax.lax.dot_general(
        xb, w, (((1,), (1,)), ((), ())), preferred_element_type=jnp.float32
    )
    x2 = jnp.sum(xe * xe, axis=1, keepdims=True)
    c2 = jnp.sum(c * c, axis=1)[None, :]
    xc = jax.lax.dot_general(
        xe, c, (((1,), (1,)), ((), ())), preferred_element_type=jnp.float32
    )
    d2 = (x2 + c2) - 2.0 * xc
    dmin = jnp.min(d2, axis=1, keepdims=True)
    out_ref[...] = jnp.sqrt(jnp.maximum(dmin, 0.0))


@jax.jit
def kernel(x, pca_components, centroids):
    out = pl.pallas_call(
        _fused_body,
        grid=(B // BM,),
        in_specs=[
            pl.BlockSpec((BM, INPUT_DIM), lambda i: (i, 0)),
            pl.BlockSpec((EMB_DIM, INPUT_DIM), lambda i: (0, 0)),
            pl.BlockSpec((N_CLUSTERS, EMB_DIM), lambda i: (0, 0)),
        ],
        out_specs=pl.BlockSpec((BM, 1), lambda i: (i, 0)),
        out_shape=jax.ShapeDtypeStruct((B, 1), jnp.float32),
    )(x, pca_components, centroids)
    return out.reshape(B)


# speedup vs baseline: 1.3849x; 1.3849x over previous
import jax
import jax.numpy as jnp
from jax.experimental import pallas as pl

B = 16384
INPUT_DIM = 512
EMB_DIM = 128
N_CLUSTERS = 64

BM = 1024
NB = B // BM


def _fused_body(x_ref, w_ref, c_ref, out_ref):
    xb = x_ref[...]
    w = w_ref[...]
    c = c_ref[...]
    xeT = jax.lax.dot_general(
        w, xb, (((1,), (1,)), ((), ())), preferred_element_type=jnp.float32
    )
    x2 = jnp.sum(xeT * xeT, axis=0, keepdims=True)
    c2 = jnp.sum(c * c, axis=1, keepdims=True)
    xcT = jax.lax.dot_general(
        c, xeT, (((1,), (0,)), ((), ())), preferred_element_type=jnp.float32
    )
    part = c2 - 2.0 * xcT
    m = jnp.min(part, axis=0, keepdims=True)
    out_ref[...] = jnp.sqrt(jnp.maximum(m + x2, 0.0))[None]


@jax.jit
def kernel(x, pca_components, centroids):
    out = pl.pallas_call(
        _fused_body,
        grid=(NB,),
        in_specs=[
            pl.BlockSpec((BM, INPUT_DIM), lambda i: (i, 0)),
            pl.BlockSpec((EMB_DIM, INPUT_DIM), lambda i: (0, 0)),
            pl.BlockSpec((N_CLUSTERS, EMB_DIM), lambda i: (0, 0)),
        ],
        out_specs=pl.BlockSpec((1, 1, BM), lambda i: (i, 0, 0)),
        out_shape=jax.ShapeDtypeStruct((NB, 1, BM), jnp.float32),
    )(x, pca_components, centroids)
    return out.reshape(B)


# speedup vs baseline: 1.8329x; 1.3235x over previous
import jax
import jax.numpy as jnp
from jax.experimental import pallas as pl
from jax.experimental.pallas import tpu as pltpu

B = 16384
INPUT_DIM = 512
EMB_DIM = 128
N_CLUSTERS = 64

BM = 2048
NB = B // BM


def _fused_body(x_ref, w_ref, c_ref, out_ref):
    xb = x_ref[...]
    w = w_ref[...]
    c = c_ref[...]
    xeT = jax.lax.dot_general(
        w, xb, (((1,), (1,)), ((), ())), preferred_element_type=jnp.float32
    )
    x2 = jnp.sum(xeT * xeT, axis=0, keepdims=True)
    c2 = jnp.sum(c * c, axis=1, keepdims=True)
    xcT = jax.lax.dot_general(
        c, xeT, (((1,), (0,)), ((), ())), preferred_element_type=jnp.float32
    )
    part = c2 - 2.0 * xcT
    m = jnp.min(part, axis=0, keepdims=True)
    out_ref[...] = jnp.sqrt(jnp.maximum(m + x2, 0.0))[None]


@jax.jit
def kernel(x, pca_components, centroids):
    out = pl.pallas_call(
        _fused_body,
        grid=(NB,),
        in_specs=[
            pl.BlockSpec((BM, INPUT_DIM), lambda i: (i, 0)),
            pl.BlockSpec((EMB_DIM, INPUT_DIM), lambda i: (0, 0)),
            pl.BlockSpec((N_CLUSTERS, EMB_DIM), lambda i: (0, 0)),
        ],
        out_specs=pl.BlockSpec((1, 1, BM), lambda i: (i, 0, 0)),
        out_shape=jax.ShapeDtypeStruct((NB, 1, BM), jnp.float32),
        compiler_params=pltpu.CompilerParams(
            dimension_semantics=("parallel",),
        ),
    )(x, pca_components, centroids)
    return out.reshape(B)


# speedup vs baseline: 2.0400x; 1.1130x over previous
import jax
import jax.numpy as jnp
from jax.experimental import pallas as pl
from jax.experimental.pallas import tpu as pltpu

B = 16384
INPUT_DIM = 512
EMB_DIM = 128
N_CLUSTERS = 64

BM = 4096
NB = B // BM


def _fused_body(x_ref, w_ref, c_ref, out_ref):
    xb = x_ref[...]
    w = w_ref[...]
    c = c_ref[...]
    xeT = jax.lax.dot_general(
        w, xb, (((1,), (1,)), ((), ())), preferred_element_type=jnp.float32
    )
    x2 = jnp.sum(xeT * xeT, axis=0, keepdims=True)
    c2 = jnp.sum(c * c, axis=1, keepdims=True)
    xcT = jax.lax.dot_general(
        c, xeT, (((1,), (0,)), ((), ())), preferred_element_type=jnp.float32
    )
    part = c2 - 2.0 * xcT
    m = jnp.min(part, axis=0, keepdims=True)
    out_ref[...] = jnp.sqrt(jnp.maximum(m + x2, 0.0))[None]


@jax.jit
def kernel(x, pca_components, centroids):
    out = pl.pallas_call(
        _fused_body,
        grid=(NB,),
        in_specs=[
            pl.BlockSpec((BM, INPUT_DIM), lambda i: (i, 0)),
            pl.BlockSpec((EMB_DIM, INPUT_DIM), lambda i: (0, 0)),
            pl.BlockSpec((N_CLUSTERS, EMB_DIM), lambda i: (0, 0)),
        ],
        out_specs=pl.BlockSpec((1, 1, BM), lambda i: (i, 0, 0)),
        out_shape=jax.ShapeDtypeStruct((NB, 1, BM), jnp.float32),
        compiler_params=pltpu.CompilerParams(
            dimension_semantics=("parallel",),
        ),
    )(x, pca_components, centroids)
    return out.reshape(B)
